# baseline (speedup 1.0000x reference)
import functools
import math

import jax
import jax.numpy as jnp
from jax import lax
from jax.experimental import pallas as pl
from jax.experimental.pallas import tpu as pltpu
from jax.experimental.pallas import tpu_sc as plsc

N0 = 10000
E = 320000
C = 64
CLASSES = 40
DEPTH = 3

NLVL = [10000, 5000, 2500, 1250]
TAB = [10112, 5120, 2560, 1280]
RB = [t // 128 for t in TAB]

NC, NS = 2, 16
NW = NC * NS
EPT = 10240
E_PAD = EPT * NW
ECH = EPT // 128
NBUF = 8

_MESH = plsc.VectorSubcoreMesh(
    core_axis_name="c", subcore_axis_name="s", num_cores=NC, num_subcores=NS)

F32 = jnp.float32
I32 = jnp.int32


def _z():
  return jnp.int32(0)

_SC_PARAMS = pltpu.CompilerParams(needs_layout_passes=False, use_tc_tiling_on_sc=False)


def _wid():
  return lax.axis_index("s") * NC + lax.axis_index("c")


def _mm_y_body(x_ref, w_ref, dinv_ref, xw_ref, y_ref):
  xw = jnp.dot(x_ref[...], w_ref[...], preferred_element_type=F32)
  xw_ref[...] = xw
  y_ref[...] = xw * dinv_ref[...]


def tc_mm_y(x, w, dinv, tab):
  r = tab // 128
  din = x.shape[1]
  return pl.pallas_call(
      _mm_y_body,
      grid=(r,),
      in_specs=[
          pl.BlockSpec((128, din), lambda i: (i, _z())),
          pl.BlockSpec((din, C), lambda i: (_z(), _z())),
          pl.BlockSpec((128, 1), lambda i: (i, _z())),
      ],
      out_specs=[
          pl.BlockSpec((128, C), lambda i: (i, _z())),
          pl.BlockSpec((128, C), lambda i: (i, _z())),
      ],
      out_shape=[
          jax.ShapeDtypeStruct((tab, C), F32),
          jax.ShapeDtypeStruct((tab, C), F32),
      ],
  )(x, w, dinv)


def _mm_y_add_body(res_ref, up_ref, w_ref, dinv_ref, xw_ref, y_ref):
  s = res_ref[...] + up_ref[...]
  xw = jnp.dot(s, w_ref[...], preferred_element_type=F32)
  xw_ref[...] = xw
  y_ref[...] = xw * dinv_ref[...]


def tc_mm_y_add(res, up, w, dinv, tab):
  r = tab // 128
  return pl.pallas_call(
      _mm_y_add_body,
      grid=(r,),
      in_specs=[
          pl.BlockSpec((128, C), lambda i: (i, _z())),
          pl.BlockSpec((128, C), lambda i: (i, _z())),
          pl.BlockSpec((C, C), lambda i: (_z(), _z())),
          pl.BlockSpec((128, 1), lambda i: (i, _z())),
      ],
      out_specs=[
          pl.BlockSpec((128, C), lambda i: (i, _z())),
          pl.BlockSpec((128, C), lambda i: (i, _z())),
      ],
      out_shape=[
          jax.ShapeDtypeStruct((tab, C), F32),
          jax.ShapeDtypeStruct((tab, C), F32),
      ],
  )(res, up, w, dinv)


_KEY_PAD = -1073741825


def _combine_body(n, relu, score, acc_ref, xw_ref, dinv_ref, b_ref, p_ref,
                  x_ref, *rest):
  pid = pl.program_id(0)
  msg = acc_ref[0] + acc_ref[1]
  dinv = dinv_ref[...]
  out = msg * dinv + xw_ref[...] * (2.0 * dinv * dinv) + b_ref[...]
  if relu:
    out = jnp.maximum(out, 0.0)
  rows = pid * 128 + lax.broadcasted_iota(I32, (128, 1), 0)
  out = jnp.where(rows < n, out, 0.0)
  x_ref[...] = out
  if score:
    score_ref, key_ref, g_ref = rest
    p = p_ref[...]
    pn = 1.0 / jnp.sqrt(jnp.sum(p * p))
    sc = jnp.tanh(jnp.dot(out, p, preferred_element_type=F32) * pn)
    score_ref[...] = sc
    bits = lax.bitcast_convert_type(sc, I32)
    key = jnp.where(bits >= 0, bits, bits ^ jnp.int32(0x7FFFFFFF))
    key_ref[...] = jnp.where(rows < n, key, jnp.int32(_KEY_PAD))
    g_ref[...] = out * sc


def tc_combine(accp, xw, dinv, b, p, n, tab, relu, score):
  r = tab // 128
  in_specs = [
      pl.BlockSpec((2, 128, C), lambda i: (_z(), i, _z())),
      pl.BlockSpec((128, C), lambda i: (i, _z())),
      pl.BlockSpec((128, 1), lambda i: (i, _z())),
      pl.BlockSpec((1, C), lambda i: (_z(), _z())),
      pl.BlockSpec((C, 1), lambda i: (_z(), _z())),
  ]
  out_specs = [pl.BlockSpec((128, C), lambda i: (i, _z()))]
  out_shape = [jax.ShapeDtypeStruct((tab, C), F32)]
  if score:
    out_specs += [
        pl.BlockSpec((128, 1), lambda i: (i, _z())),
        pl.BlockSpec((128, 1), lambda i: (i, _z())),
        pl.BlockSpec((128, C), lambda i: (i, _z())),
    ]
    out_shape += [
        jax.ShapeDtypeStruct((tab, 1), F32),
        jax.ShapeDtypeStruct((tab, 1), I32),
        jax.ShapeDtypeStruct((tab, C), F32),
    ]
  return pl.pallas_call(
      functools.partial(_combine_body, n, relu, score),
      grid=(r,),
      in_specs=in_specs,
      out_specs=out_specs,
      out_shape=out_shape,
  )(accp, xw, dinv, b, p)


def _select_body(k, kf_ref, krow_ref, map_ref, sm_i, sm_f):
  pid = pl.program_id(0)
  imin = jnp.int32(-2147483648)

  @pl.when(pid == 0)
  def _bisect():
    kf = kf_ref[...]

    def step(it, bacc):
      b = (31 - it).astype(I32)
      cand = bacc | lax.shift_left(jnp.int32(1), b)
      cnt = jnp.sum((kf >= (cand ^ imin)).astype(F32))
      return jnp.where(cnt >= F32(k), cand, bacc)

    bfin = lax.fori_loop(jnp.int32(0), jnp.int32(32), step, jnp.int32(0))
    t = bfin ^ imin
    c1 = jnp.sum((kf > t).astype(F32))
    sm_i[0] = t
    sm_i[1] = jnp.int32(k) - c1.astype(I32)
    sm_f[0] = 0.0
    sm_f[1] = 0.0

  t = sm_i[0]
  budget = sm_i[1].astype(F32)
  row = krow_ref[0]
  strict = (row > t).astype(F32)
  tie = (row == t).astype(F32)
  m2 = jnp.concatenate([strict, tie], axis=0)
  ia = lax.broadcasted_iota(I32, (128, 128), 0)
  ib = lax.broadcasted_iota(I32, (128, 128), 1)
  u = (ia < ib).astype(F32)
  pref = jnp.dot(m2, u, preferred_element_type=F32)
  ps = pref[0:1, :] + sm_f[0]
  pt = pref[1:2, :] + sm_f[1]
  chosen = (strict > 0.0) | ((tie > 0.0) & (pt < budget))
  mp = ps + jnp.minimum(pt, budget)
  map_ref[0] = jnp.where(chosen, mp.astype(I32), jnp.int32(-1))
  sm_f[0] = sm_f[0] + jnp.sum(strict)
  sm_f[1] = sm_f[1] + jnp.sum(tie)


def tc_select(keys2d, k, tab):
  r = tab // 128
  return pl.pallas_call(
      functools.partial(_select_body, k),
      grid=(r,),
      in_specs=[
          pl.BlockSpec((r, 1, 128), lambda i: (_z(), _z(), _z())),
          pl.BlockSpec((1, 1, 128), lambda i: (i, _z(), _z())),
      ],
      out_specs=pl.BlockSpec((1, 1, 128), lambda i: (i, _z(), _z())),
      out_shape=jax.ShapeDtypeStruct((r, 1, 128), I32),
      scratch_shapes=[
          pltpu.SMEM((2,), I32),
          pltpu.SMEM((2,), F32),
      ],
  )(keys2d, keys2d)


def _deg_reduce_body(degp_ref, dinv_ref):
  deg = jnp.sum(degp_ref[...], axis=0, keepdims=True) + 2.0
  dinv_ref[0] = 1.0 / jnp.sqrt(deg)


def tc_deg_reduce(degp, tab):
  r = tab // 128
  return pl.pallas_call(
      _deg_reduce_body,
      grid=(r,),
      in_specs=[pl.BlockSpec((NW, 128), lambda i: (_z(), i))],
      out_specs=pl.BlockSpec((1, 1, 128), lambda i: (i, _z(), _z())),
      out_shape=jax.ShapeDtypeStruct((r, 1, 128), F32),
  )(degp)


def _final_body(z_ref, o_ref):
  logits = z_ref[...][:, :CLASSES]
  m = jnp.max(logits, axis=1, keepdims=True)
  s = jnp.sum(jnp.exp(logits - m), axis=1, keepdims=True)
  o_ref[...] = logits - m - jnp.log(s)


def tc_final(z):
  return pl.pallas_call(
      _final_body,
      grid=(125,),
      in_specs=[pl.BlockSpec((80, C), lambda i: (i, _z()))],
      out_specs=pl.BlockSpec((80, CLASSES), lambda i: (i, _z())),
      out_shape=jax.ShapeDtypeStruct((N0, CLASSES), F32),
  )(z)


def sc_deg(dstm, tab):
  def body(dst_hbm, degp_hbm, dbuf, degv):
    w = _wid()
    base = pl.multiple_of(w * EPT, 128)
    pltpu.sync_copy(dst_hbm.at[pl.ds(base, EPT)], dbuf)
    ones = jnp.ones((16,), F32)
    zeros = jnp.zeros((16,), F32)

    def zstep(i, _):
      degv[pl.ds(i * 16, 16)] = zeros
      return jnp.int32(0)

    lax.fori_loop(jnp.int32(0), jnp.int32(tab // 16), zstep, jnp.int32(0))

    def estep(i, _):
      idx = dbuf[pl.ds(i * 16, 16)]
      plsc.addupdate_scatter(degv, [idx], ones)
      return jnp.int32(0)

    lax.fori_loop(jnp.int32(0), jnp.int32(EPT // 16), estep, jnp.int32(0))
    pltpu.sync_copy(degv, degp_hbm.at[w])

  return pl.kernel(
      body,
      out_type=jax.ShapeDtypeStruct((NW, tab), F32),
      mesh=_MESH,
      compiler_params=_SC_PARAMS,
      scratch_types=[
          pltpu.VMEM((EPT,), I32),
          pltpu.VMEM((tab,), F32),
      ],
  )(dstm)


def sc_msg(src, dstm, y, zeros_tab, tab):
  def body(src_hbm, dst_hbm, y_hbm, z_hbm, accp_hbm, sidx, didx, dbufs, rows,
           acc, gsem):
    cid = lax.axis_index("c")
    sid = lax.axis_index("s")
    w = _wid()
    ebase = pl.multiple_of(w * EPT, 128)
    pltpu.sync_copy(src_hbm.at[pl.ds(ebase, EPT)], sidx)
    pltpu.sync_copy(dst_hbm.at[pl.ds(ebase, EPT)], didx)
    rz = tab // NS
    zbase = pl.multiple_of(sid * rz, 8)
    pltpu.sync_copy(z_hbm.at[pl.ds(zbase, rz)], acc.at[pl.ds(zbase, rz)])
    plsc.subcore_barrier()

    def gather(ch, b):
      bi = jnp.int32(b)
      return pltpu.make_async_copy(
          y_hbm.at[sidx.at[pl.ds(ch * 128, 128)]], rows.at[bi], gsem.at[bi])

    for b in range(NBUF):
      gather(jnp.int32(b), b).start()

    def round_(r, _):
      for b in range(NBUF):
        ch = r * NBUF + b
        gather(ch, b).wait()
        for j in range(8):
          dbufs[b, pl.ds(j * 16, 16)] = didx[pl.ds(ch * 128 + j * 16, 16)]
        pltpu.sync_copy(rows.at[jnp.int32(b)], acc.at[dbufs.at[jnp.int32(b)]], add=True)

        @pl.when(ch + NBUF < ECH)
        def _next():
          gather(ch + NBUF, b).start()

      return jnp.int32(0)

    lax.fori_loop(jnp.int32(0), jnp.int32(ECH // NBUF), round_, jnp.int32(0))
    plsc.subcore_barrier()
    pltpu.sync_copy(acc.at[pl.ds(zbase, rz)],
                    accp_hbm.at[cid, pl.ds(zbase, rz)])

  return pl.kernel(
      body,
      out_type=jax.ShapeDtypeStruct((NC, tab, C), F32),
      mesh=_MESH,
      compiler_params=_SC_PARAMS,
      scratch_types=[
          pltpu.VMEM((EPT,), I32),
          pltpu.VMEM((EPT,), I32),
          pltpu.VMEM((NBUF, 128), I32),
          pltpu.VMEM((NBUF, 128, C), F32),
          pltpu.VMEM_SHARED((tab, C), F32),
          pltpu.SemaphoreType.DMA((NBUF,)),
      ],
  )(src, dstm, y, zeros_tab)


def sc_remap(src, dstm, mapping, g, tab_l, tab_n, kn):
  rchunks = tab_l // 128
  rounds = (rchunks + NW - 1) // NW

  def body(src_hbm, dst_hbm, map_hbm, g_hbm, nsrc_hbm, ndst_hbm, degp_hbm,
           xp_hbm, mapv, sidx, didx, osx, odx, ibuf, rows, degv, sem):
    w = _wid()
    ebase = pl.multiple_of(w * EPT, 128)
    pltpu.sync_copy(map_hbm, mapv)
    pltpu.sync_copy(src_hbm.at[pl.ds(ebase, EPT)], sidx)
    pltpu.sync_copy(dst_hbm.at[pl.ds(ebase, EPT)], didx)
    ones = jnp.ones((16,), F32)
    zeros = jnp.zeros((16,), F32)

    def zstep(i, _):
      degv[pl.ds(i * 16, 16)] = zeros
      return jnp.int32(0)

    lax.fori_loop(jnp.int32(0), jnp.int32(tab_n // 16), zstep, jnp.int32(0))

    def estep(i, _):
      sv = sidx[pl.ds(i * 16, 16)]
      dv = didx[pl.ds(i * 16, 16)]
      ms = plsc.load_gather(mapv, [sv])
      md = plsc.load_gather(mapv, [dv])
      valid = (ms >= 0) & (md >= 0)
      osx[pl.ds(i * 16, 16)] = jnp.where(valid, ms, 0)
      nd = jnp.where(valid, md, jnp.int32(kn))
      odx[pl.ds(i * 16, 16)] = nd
      plsc.addupdate_scatter(degv, [nd], ones)
      return jnp.int32(0)

    lax.fori_loop(jnp.int32(0), jnp.int32(EPT // 16), estep, jnp.int32(0))
    pltpu.sync_copy(osx, nsrc_hbm.at[pl.ds(ebase, EPT)])
    pltpu.sync_copy(odx, ndst_hbm.at[pl.ds(ebase, EPT)])
    pltpu.sync_copy(degv, degp_hbm.at[w])

    def rstep(it, _):
      ck = w + it * NW

      @pl.when(ck < rchunks)
      def _do():
        g0 = pl.multiple_of(ck * 128, 128)
        pltpu.sync_copy(g_hbm.at[pl.ds(g0, 128)], rows)
        for j in range(8):
          mv = mapv[pl.ds(g0 + j * 16, 16)]
          ibuf[pl.ds(j * 16, 16)] = jnp.where(mv >= 0, mv, jnp.int32(kn))
        pltpu.async_copy(rows, xp_hbm.at[ibuf], sem).wait()

      return jnp.int32(0)

    lax.fori_loop(jnp.int32(0), jnp.int32(rounds), rstep, jnp.int32(0))

  return pl.kernel(
      body,
      out_type=[
          jax.ShapeDtypeStruct((E_PAD,), I32),
          jax.ShapeDtypeStruct((E_PAD,), I32),
          jax.ShapeDtypeStruct((NW, tab_n), F32),
          jax.ShapeDtypeStruct((tab_n, C), F32),
      ],
      mesh=_MESH,
      compiler_params=_SC_PARAMS,
      scratch_types=[
          pltpu.VMEM((tab_l,), I32),
          pltpu.VMEM((EPT,), I32),
          pltpu.VMEM((EPT,), I32),
          pltpu.VMEM((EPT,), I32),
          pltpu.VMEM((EPT,), I32),
          pltpu.VMEM((128,), I32),
          pltpu.VMEM((128, C), F32),
          pltpu.VMEM((tab_n,), F32),
          pltpu.SemaphoreType.DMA,
      ],
  )(src, dstm, mapping, g)


def sc_upgather(mapping, xnext, tab_j, tab_n, dummy):
  rchunks = tab_j // 128
  rounds = (rchunks + NW - 1) // NW

  def body(map_hbm, xn_hbm, up_hbm, mbuf, ibuf, rows, sem):
    w = _wid()

    def rstep(it, _):
      ck = w + it * NW

      @pl.when(ck < rchunks)
      def _do():
        g0 = pl.multiple_of(ck * 128, 128)
        pltpu.sync_copy(map_hbm.at[pl.ds(g0, 128)], mbuf)
        for j in range(8):
          mv = mbuf[pl.ds(j * 16, 16)]
          ibuf[pl.ds(j * 16, 16)] = jnp.where(mv >= 0, mv, jnp.int32(dummy))
        pltpu.async_copy(xn_hbm.at[ibuf], rows, sem).wait()
        pltpu.sync_copy(rows, up_hbm.at[pl.ds(g0, 128)])

      return jnp.int32(0)

    lax.fori_loop(jnp.int32(0), jnp.int32(rounds), rstep, jnp.int32(0))

  return pl.kernel(
      body,
      out_type=jax.ShapeDtypeStruct((tab_j, C), F32),
      mesh=_MESH,
      compiler_params=_SC_PARAMS,
      scratch_types=[
          pltpu.VMEM((128,), I32),
          pltpu.VMEM((128,), I32),
          pltpu.VMEM((128, C), F32),
          pltpu.SemaphoreType.DMA,
      ],
  )(mapping, xnext)


def kernel(x, edge_index, enc_W0, enc_b0, enc_W1, enc_b1, enc_W2, enc_b2,
           enc_W3, enc_b3, pool_p0, pool_p1, pool_p2, dec_W0, dec_b0, dec_W1,
           dec_b1, dec_W2, dec_b2):
  eW = [enc_W0, enc_W1, enc_W2, enc_W3]
  eB = [enc_b0, enc_b1, enc_b2, enc_b3]
  pP = [pool_p0, pool_p1, pool_p2]
  w2p = jnp.zeros((C, C), F32).at[:, :CLASSES].set(dec_W2)
  b2p = jnp.zeros((C,), F32).at[:CLASSES].set(dec_b2)
  dW = [dec_W0, dec_W1, w2p]
  dB = [dec_b0, dec_b1, b2p]

  src0 = edge_index[0].astype(I32)
  dst0 = edge_index[1].astype(I32)
  src0 = jnp.concatenate(
      [src0, jnp.zeros((E_PAD - E,), I32)])
  dst0 = jnp.concatenate(
      [dst0, jnp.full((E_PAD - E,), NLVL[0], I32)])

  xin = jnp.pad(x.astype(F32), ((0, TAB[0] - N0), (0, 0)))
  zeros_tab = jnp.zeros((TAB[0], C), F32)

  def dinv_of(degp, tab):
    return tc_deg_reduce(degp, tab).reshape(tab, 1)

  dinv = [None] * 4
  dinv[0] = dinv_of(sc_deg(dst0, TAB[0]), TAB[0])
  edges = [(src0, dst0)] + [None] * 3
  xw, y = tc_mm_y(xin, eW[0], dinv[0], TAB[0])
  accp = sc_msg(src0, dst0, y, zeros_tab, TAB[0])

  xs = [None] * 3
  mappings = [None] * 3
  xcur = None
  for l in range(DEPTH):
    n, tab = NLVL[l], TAB[l]
    kn, tab_n = NLVL[l + 1], TAB[l + 1]
    xcur, _, keys, g = tc_combine(
        accp, xw, dinv[l], eB[l].reshape(1, C), pP[l].reshape(C, 1),
        n, tab, relu=True, score=True)
    xs[l] = xcur
    mapping = tc_select(keys.reshape(tab // 128, 1, 128), kn, tab).reshape(tab)
    mappings[l] = mapping
    s_l, d_l = edges[l]
    nsrc, ndst, degp, xp = sc_remap(s_l, d_l, mapping, g, tab, tab_n, kn)
    edges[l + 1] = (nsrc, ndst)
    dinv[l + 1] = dinv_of(degp, tab_n)
    xw, y = tc_mm_y(xp, eW[l + 1], dinv[l + 1], tab_n)
    accp = sc_msg(nsrc, ndst, y, zeros_tab[:tab_n], tab_n)

  xcur = tc_combine(
      accp, xw, dinv[DEPTH], eB[DEPTH].reshape(1, C),
      pP[0].reshape(C, 1), NLVL[DEPTH], TAB[DEPTH], relu=True, score=False)[0]

  for i in range(DEPTH):
    j = DEPTH - 1 - i
    n, tab = NLVL[j], TAB[j]
    up = sc_upgather(mappings[j], xcur, tab, TAB[j + 1], NLVL[j + 1])
    xw, y = tc_mm_y_add(xs[j], up, dW[i], dinv[j], tab)
    s_l, d_l = edges[j]
    accp = sc_msg(s_l, d_l, y, zeros_tab[:tab], tab)
    xcur = tc_combine(
        accp, xw, dinv[j], dB[i].reshape(1, C), pP[0].reshape(C, 1),
        n, tab, relu=(i < DEPTH - 1), score=False)[0]

  return tc_final(xcur)


# speedup vs baseline: 18.4818x; 18.4818x over previous
import functools
import math

import jax
import jax.numpy as jnp
from jax import lax
from jax.experimental import pallas as pl
from jax.experimental.pallas import tpu as pltpu
from jax.experimental.pallas import tpu_sc as plsc

N0 = 10000
E = 320000
C = 64
CLASSES = 40
DEPTH = 3

NLVL = [10000, 5000, 2500, 1250]
TAB = [10112, 5120, 2560, 1280]
RB = [t // 128 for t in TAB]

NC, NS = 2, 16
NW = NC * NS
EPT = 10240
E_PAD = EPT * NW
ECH = EPT // 128
NBUF = 8

_MESH = plsc.VectorSubcoreMesh(
    core_axis_name="c", subcore_axis_name="s", num_cores=NC, num_subcores=NS)

F32 = jnp.float32
I32 = jnp.int32


def _z():
  return jnp.int32(0)

_SC_PARAMS = pltpu.CompilerParams(needs_layout_passes=False, use_tc_tiling_on_sc=False)


def _wid():
  return lax.axis_index("s") * NC + lax.axis_index("c")


def _nchunks(didx, n):
  total = jnp.float32(0.0)
  for q in range(ECH // 16):
    pos = lax.iota(I32, 16) * 128 + q * 2048
    v = plsc.load_gather(didx, [pos])
    total = total + jnp.sum((v < n).astype(F32))
  return total.astype(I32)


def _mm_y_body(x_ref, w_ref, dinv_ref, xw_ref, y_ref):
  xw = jnp.dot(x_ref[...], w_ref[...], preferred_element_type=F32)
  xw_ref[...] = xw
  y_ref[...] = xw * dinv_ref[...]


def tc_mm_y(x, w, dinv, tab):
  r = tab // 128
  din = x.shape[1]
  return pl.pallas_call(
      _mm_y_body,
      grid=(r,),
      in_specs=[
          pl.BlockSpec((128, din), lambda i: (i, _z())),
          pl.BlockSpec((din, C), lambda i: (_z(), _z())),
          pl.BlockSpec((128, 1), lambda i: (i, _z())),
      ],
      out_specs=[
          pl.BlockSpec((128, C), lambda i: (i, _z())),
          pl.BlockSpec((128, C), lambda i: (i, _z())),
      ],
      out_shape=[
          jax.ShapeDtypeStruct((tab, C), F32),
          jax.ShapeDtypeStruct((tab, C), F32),
      ],
  )(x, w, dinv)


def _mm_y_add_body(res_ref, up_ref, w_ref, dinv_ref, xw_ref, y_ref):
  s = res_ref[...] + up_ref[...]
  xw = jnp.dot(s, w_ref[...], preferred_element_type=F32)
  xw_ref[...] = xw
  y_ref[...] = xw * dinv_ref[...]


def tc_mm_y_add(res, up, w, dinv, tab):
  r = tab // 128
  return pl.pallas_call(
      _mm_y_add_body,
      grid=(r,),
      in_specs=[
          pl.BlockSpec((128, C), lambda i: (i, _z())),
          pl.BlockSpec((128, C), lambda i: (i, _z())),
          pl.BlockSpec((C, C), lambda i: (_z(), _z())),
          pl.BlockSpec((128, 1), lambda i: (i, _z())),
      ],
      out_specs=[
          pl.BlockSpec((128, C), lambda i: (i, _z())),
          pl.BlockSpec((128, C), lambda i: (i, _z())),
      ],
      out_shape=[
          jax.ShapeDtypeStruct((tab, C), F32),
          jax.ShapeDtypeStruct((tab, C), F32),
      ],
  )(res, up, w, dinv)


_KEY_PAD = -1073741825


def _combine_body(n, relu, score, acc_ref, xw_ref, dinv_ref, b_ref, p_ref,
                  x_ref, *rest):
  pid = pl.program_id(0)
  msg = acc_ref[0] + acc_ref[1]
  dinv = dinv_ref[...]
  out = msg * dinv + xw_ref[...] * (2.0 * dinv * dinv) + b_ref[...]
  if relu:
    out = jnp.maximum(out, 0.0)
  rows = pid * 128 + lax.broadcasted_iota(I32, (128, 1), 0)
  out = jnp.where(rows < n, out, 0.0)
  x_ref[...] = out
  if score:
    score_ref, key_ref, g_ref = rest
    p = p_ref[...]
    pn = 1.0 / jnp.sqrt(jnp.sum(p * p))
    sc = jnp.tanh(jnp.dot(out, p, preferred_element_type=F32) * pn)
    score_ref[...] = sc
    bits = lax.bitcast_convert_type(sc, I32)
    key = jnp.where(bits >= 0, bits, bits ^ jnp.int32(0x7FFFFFFF))
    key_ref[...] = jnp.where(rows < n, key, jnp.int32(_KEY_PAD))
    g_ref[...] = out * sc


def tc_combine(accp, xw, dinv, b, p, n, tab, relu, score):
  r = tab // 128
  in_specs = [
      pl.BlockSpec((2, 128, C), lambda i: (_z(), i, _z())),
      pl.BlockSpec((128, C), lambda i: (i, _z())),
      pl.BlockSpec((128, 1), lambda i: (i, _z())),
      pl.BlockSpec((1, C), lambda i: (_z(), _z())),
      pl.BlockSpec((C, 1), lambda i: (_z(), _z())),
  ]
  out_specs = [pl.BlockSpec((128, C), lambda i: (i, _z()))]
  out_shape = [jax.ShapeDtypeStruct((tab, C), F32)]
  if score:
    out_specs += [
        pl.BlockSpec((128, 1), lambda i: (i, _z())),
        pl.BlockSpec((128, 1), lambda i: (i, _z())),
        pl.BlockSpec((128, C), lambda i: (i, _z())),
    ]
    out_shape += [
        jax.ShapeDtypeStruct((tab, 1), F32),
        jax.ShapeDtypeStruct((tab, 1), I32),
        jax.ShapeDtypeStruct((tab, C), F32),
    ]
  return pl.pallas_call(
      functools.partial(_combine_body, n, relu, score),
      grid=(r,),
      in_specs=in_specs,
      out_specs=out_specs,
      out_shape=out_shape,
  )(accp, xw, dinv, b, p)


def _select_body(k, kf_ref, krow_ref, map_ref, sm_i, sm_f):
  pid = pl.program_id(0)
  imin = jnp.int32(-2147483648)

  @pl.when(pid == 0)
  def _bisect():
    kf = kf_ref[...]

    def step(it, bacc):
      b = (31 - it).astype(I32)
      cand = bacc | lax.shift_left(jnp.int32(1), b)
      cnt = jnp.sum((kf >= (cand ^ imin)).astype(F32))
      return jnp.where(cnt >= F32(k), cand, bacc)

    bfin = lax.fori_loop(jnp.int32(0), jnp.int32(32), step, jnp.int32(0))
    t = bfin ^ imin
    c1 = jnp.sum((kf > t).astype(F32))
    sm_i[0] = t
    sm_i[1] = jnp.int32(k) - c1.astype(I32)
    sm_f[0] = 0.0
    sm_f[1] = 0.0

  t = sm_i[0]
  budget = sm_i[1].astype(F32)
  row = krow_ref[0]
  strict = (row > t).astype(F32)
  tie = (row == t).astype(F32)
  m2 = jnp.concatenate([strict, tie], axis=0)
  ia = lax.broadcasted_iota(I32, (128, 128), 0)
  ib = lax.broadcasted_iota(I32, (128, 128), 1)
  u = (ia < ib).astype(F32)
  pref = jnp.dot(m2, u, preferred_element_type=F32)
  ps = pref[0:1, :] + sm_f[0]
  pt = pref[1:2, :] + sm_f[1]
  chosen = (strict > 0.0) | ((tie > 0.0) & (pt < budget))
  mp = ps + jnp.minimum(pt, budget)
  map_ref[0] = jnp.where(chosen, mp.astype(I32), jnp.int32(-1))
  sm_f[0] = sm_f[0] + jnp.sum(strict)
  sm_f[1] = sm_f[1] + jnp.sum(tie)


def tc_select(keys2d, k, tab):
  r = tab // 128
  return pl.pallas_call(
      functools.partial(_select_body, k),
      grid=(r,),
      in_specs=[
          pl.BlockSpec((r, 1, 128), lambda i: (_z(), _z(), _z())),
          pl.BlockSpec((1, 1, 128), lambda i: (i, _z(), _z())),
      ],
      out_specs=pl.BlockSpec((1, 1, 128), lambda i: (i, _z(), _z())),
      out_shape=jax.ShapeDtypeStruct((r, 1, 128), I32),
      scratch_shapes=[
          pltpu.SMEM((2,), I32),
          pltpu.SMEM((2,), F32),
      ],
  )(keys2d, keys2d)


def _deg_reduce_body(degp_ref, dinv_ref):
  deg = jnp.sum(degp_ref[...], axis=0, keepdims=True) + 2.0
  dinv_ref[0] = 1.0 / jnp.sqrt(deg)


def tc_deg_reduce(degp, tab):
  r = tab // 128
  return pl.pallas_call(
      _deg_reduce_body,
      grid=(r,),
      in_specs=[pl.BlockSpec((NW, 128), lambda i: (_z(), i))],
      out_specs=pl.BlockSpec((1, 1, 128), lambda i: (i, _z(), _z())),
      out_shape=jax.ShapeDtypeStruct((r, 1, 128), F32),
  )(degp)


def _final_body(z_ref, o_ref):
  logits = z_ref[...][:, :CLASSES]
  m = jnp.max(logits, axis=1, keepdims=True)
  s = jnp.sum(jnp.exp(logits - m), axis=1, keepdims=True)
  o_ref[...] = logits - m - jnp.log(s)


def tc_final(z):
  return pl.pallas_call(
      _final_body,
      grid=(125,),
      in_specs=[pl.BlockSpec((80, C), lambda i: (i, _z()))],
      out_specs=pl.BlockSpec((80, CLASSES), lambda i: (i, _z())),
      out_shape=jax.ShapeDtypeStruct((N0, CLASSES), F32),
  )(z)


def sc_deg(dstm, tab):
  def body(dst_hbm, degp_hbm, dbuf, degv):
    w = _wid()
    base = pl.multiple_of(w * EPT, 128)
    pltpu.sync_copy(dst_hbm.at[pl.ds(base, EPT)], dbuf)
    ones = jnp.ones((16,), F32)
    zeros = jnp.zeros((16,), F32)

    def zstep(i, _):
      degv[pl.ds(i * 16, 16)] = zeros
      return jnp.int32(0)

    lax.fori_loop(jnp.int32(0), jnp.int32(tab // 16), zstep, jnp.int32(0))

    def estep(i, _):
      idx = dbuf[pl.ds(i * 16, 16)]
      plsc.addupdate_scatter(degv, [idx], ones)
      return jnp.int32(0)

    lax.fori_loop(jnp.int32(0), jnp.int32(EPT // 16), estep, jnp.int32(0))
    pltpu.sync_copy(degv, degp_hbm.at[w])

  return pl.kernel(
      body,
      out_type=jax.ShapeDtypeStruct((NW, tab), F32),
      mesh=_MESH,
      compiler_params=_SC_PARAMS,
      scratch_types=[
          pltpu.VMEM((EPT,), I32),
          pltpu.VMEM((tab,), F32),
      ],
  )(dstm)


def sc_msg(src, dstm, y, zeros_tab, tab, nvalid):
  def body(src_hbm, dst_hbm, y_hbm, z_hbm, accp_hbm, sidx, didx, dbufs, rows,
           acc, gsem):
    cid = lax.axis_index("c")
    sid = lax.axis_index("s")
    w = _wid()
    ebase = pl.multiple_of(w * EPT, 128)
    pltpu.sync_copy(src_hbm.at[pl.ds(ebase, EPT)], sidx)
    pltpu.sync_copy(dst_hbm.at[pl.ds(ebase, EPT)], didx)
    rz = tab // NS
    zbase = pl.multiple_of(sid * rz, 8)
    pltpu.sync_copy(z_hbm.at[pl.ds(zbase, rz)], acc.at[pl.ds(zbase, rz)])
    plsc.subcore_barrier()
    nch = _nchunks(didx, nvalid)

    def gather(ch, b):
      bi = jnp.int32(b)
      return pltpu.make_async_copy(
          y_hbm.at[sidx.at[pl.ds(ch * 128, 128)]], rows.at[bi], gsem.at[bi])

    for b in range(NBUF):

      @pl.when(jnp.int32(b) < nch)
      def _prime():
        gather(jnp.int32(b), b).start()

    def round_(r, _):
      for b in range(NBUF):
        ch = r * NBUF + b

        @pl.when(ch < nch)
        def _work():
          gather(ch, b).wait()
          for j in range(8):
            dbufs[b, pl.ds(j * 16, 16)] = didx[pl.ds(ch * 128 + j * 16, 16)]
          pltpu.sync_copy(rows.at[jnp.int32(b)], acc.at[dbufs.at[jnp.int32(b)]],
                          add=True)

          @pl.when(ch + NBUF < nch)
          def _next():
            gather(ch + NBUF, b).start()

      return jnp.int32(0)

    rounds = lax.div(nch + jnp.int32(NBUF - 1), jnp.int32(NBUF))
    lax.fori_loop(jnp.int32(0), rounds, round_, jnp.int32(0))
    plsc.subcore_barrier()
    pltpu.sync_copy(acc.at[pl.ds(zbase, rz)],
                    accp_hbm.at[cid, pl.ds(zbase, rz)])

  return pl.kernel(
      body,
      out_type=jax.ShapeDtypeStruct((NC, tab, C), F32),
      mesh=_MESH,
      compiler_params=_SC_PARAMS,
      scratch_types=[
          pltpu.VMEM((EPT,), I32),
          pltpu.VMEM((EPT,), I32),
          pltpu.VMEM((NBUF, 128), I32),
          pltpu.VMEM((NBUF, 128, C), F32),
          pltpu.VMEM_SHARED((tab, C), F32),
          pltpu.SemaphoreType.DMA((NBUF,)),
      ],
  )(src, dstm, y, zeros_tab)


def sc_remap(src, dstm, mapping, g, tab_l, tab_n, kn, nvalid):
  rchunks = tab_l // 128
  rounds = (rchunks + NW - 1) // NW

  def body(src_hbm, dst_hbm, map_hbm, g_hbm, nsrc_hbm, ndst_hbm, degp_hbm,
           xp_hbm, mapv, sidx, didx, osx, odx, ibuf, rows, degv, sem):
    w = _wid()
    ebase = pl.multiple_of(w * EPT, 128)
    pltpu.sync_copy(map_hbm, mapv)
    pltpu.sync_copy(src_hbm.at[pl.ds(ebase, EPT)], sidx)
    pltpu.sync_copy(dst_hbm.at[pl.ds(ebase, EPT)], didx)
    ones = jnp.ones((16,), F32)
    zeros = jnp.zeros((16,), F32)
    zeros_i = jnp.zeros((16,), I32)
    dummy_i = jnp.full((16,), kn, I32)

    def zstep(i, _):
      degv[pl.ds(i * 16, 16)] = zeros
      return jnp.int32(0)

    lax.fori_loop(jnp.int32(0), jnp.int32(tab_n // 16), zstep, jnp.int32(0))

    def pstep(i, _):
      osx[pl.ds(i * 16, 16)] = zeros_i
      odx[pl.ds(i * 16, 16)] = dummy_i
      return jnp.int32(0)

    lax.fori_loop(jnp.int32(0), jnp.int32(EPT // 16), pstep, jnp.int32(0))
    ngroups = _nchunks(didx, nvalid) * 8

    def estep(i, cnt):
      sv = sidx[pl.ds(i * 16, 16)]
      dv = didx[pl.ds(i * 16, 16)]
      ms = plsc.load_gather(mapv, [sv])
      md = plsc.load_gather(mapv, [dv])
      valid = (ms >= 0) & (md >= 0)
      plsc.store_compressed(osx.at[pl.ds(cnt, 16)], ms, mask=valid)
      plsc.store_compressed(odx.at[pl.ds(cnt, 16)], md, mask=valid)
      plsc.addupdate_scatter(degv, [md], ones, mask=valid)
      return cnt + jnp.sum(valid.astype(F32)).astype(I32)

    lax.fori_loop(jnp.int32(0), ngroups, estep, jnp.int32(0))
    pltpu.sync_copy(osx, nsrc_hbm.at[pl.ds(ebase, EPT)])
    pltpu.sync_copy(odx, ndst_hbm.at[pl.ds(ebase, EPT)])
    pltpu.sync_copy(degv, degp_hbm.at[w])

    def rstep(it, _):
      ck = w + it * NW

      @pl.when(ck < rchunks)
      def _do():
        g0 = pl.multiple_of(ck * 128, 128)
        pltpu.sync_copy(g_hbm.at[pl.ds(g0, 128)], rows)
        for j in range(8):
          mv = mapv[pl.ds(g0 + j * 16, 16)]
          ibuf[pl.ds(j * 16, 16)] = jnp.where(mv >= 0, mv, jnp.int32(kn))
        pltpu.async_copy(rows, xp_hbm.at[ibuf], sem).wait()

      return jnp.int32(0)

    lax.fori_loop(jnp.int32(0), jnp.int32(rounds), rstep, jnp.int32(0))

  return pl.kernel(
      body,
      out_type=[
          jax.ShapeDtypeStruct((E_PAD,), I32),
          jax.ShapeDtypeStruct((E_PAD,), I32),
          jax.ShapeDtypeStruct((NW, tab_n), F32),
          jax.ShapeDtypeStruct((tab_n, C), F32),
      ],
      mesh=_MESH,
      compiler_params=_SC_PARAMS,
      scratch_types=[
          pltpu.VMEM((tab_l,), I32),
          pltpu.VMEM((EPT,), I32),
          pltpu.VMEM((EPT,), I32),
          pltpu.VMEM((EPT,), I32),
          pltpu.VMEM((EPT,), I32),
          pltpu.VMEM((128,), I32),
          pltpu.VMEM((128, C), F32),
          pltpu.VMEM((tab_n,), F32),
          pltpu.SemaphoreType.DMA,
      ],
  )(src, dstm, mapping, g)


def sc_upgather(mapping, xnext, tab_j, tab_n, dummy):
  rchunks = tab_j // 128
  rounds = (rchunks + NW - 1) // NW

  def body(map_hbm, xn_hbm, up_hbm, mbuf, ibuf, rows, sem):
    w = _wid()

    def rstep(it, _):
      ck = w + it * NW

      @pl.when(ck < rchunks)
      def _do():
        g0 = pl.multiple_of(ck * 128, 128)
        pltpu.sync_copy(map_hbm.at[pl.ds(g0, 128)], mbuf)
        for j in range(8):
          mv = mbuf[pl.ds(j * 16, 16)]
          ibuf[pl.ds(j * 16, 16)] = jnp.where(mv >= 0, mv, jnp.int32(dummy))
        pltpu.async_copy(xn_hbm.at[ibuf], rows, sem).wait()
        pltpu.sync_copy(rows, up_hbm.at[pl.ds(g0, 128)])

      return jnp.int32(0)

    lax.fori_loop(jnp.int32(0), jnp.int32(rounds), rstep, jnp.int32(0))

  return pl.kernel(
      body,
      out_type=jax.ShapeDtypeStruct((tab_j, C), F32),
      mesh=_MESH,
      compiler_params=_SC_PARAMS,
      scratch_types=[
          pltpu.VMEM((128,), I32),
          pltpu.VMEM((128,), I32),
          pltpu.VMEM((128, C), F32),
          pltpu.SemaphoreType.DMA,
      ],
  )(mapping, xnext)


def kernel(x, edge_index, enc_W0, enc_b0, enc_W1, enc_b1, enc_W2, enc_b2,
           enc_W3, enc_b3, pool_p0, pool_p1, pool_p2, dec_W0, dec_b0, dec_W1,
           dec_b1, dec_W2, dec_b2):
  eW = [enc_W0, enc_W1, enc_W2, enc_W3]
  eB = [enc_b0, enc_b1, enc_b2, enc_b3]
  pP = [pool_p0, pool_p1, pool_p2]
  w2p = jnp.zeros((C, C), F32).at[:, :CLASSES].set(dec_W2)
  b2p = jnp.zeros((C,), F32).at[:CLASSES].set(dec_b2)
  dW = [dec_W0, dec_W1, w2p]
  dB = [dec_b0, dec_b1, b2p]

  src0 = edge_index[0].astype(I32)
  dst0 = edge_index[1].astype(I32)
  src0 = jnp.concatenate(
      [src0, jnp.zeros((E_PAD - E,), I32)])
  dst0 = jnp.concatenate(
      [dst0, jnp.full((E_PAD - E,), NLVL[0], I32)])

  xin = jnp.pad(x.astype(F32), ((0, TAB[0] - N0), (0, 0)))
  zeros_tab = jnp.zeros((TAB[0], C), F32)

  def dinv_of(degp, tab):
    return tc_deg_reduce(degp, tab).reshape(tab, 1)

  dinv = [None] * 4
  dinv[0] = dinv_of(sc_deg(dst0, TAB[0]), TAB[0])
  edges = [(src0, dst0)] + [None] * 3
  xw, y = tc_mm_y(xin, eW[0], dinv[0], TAB[0])
  accp = sc_msg(src0, dst0, y, zeros_tab, TAB[0], NLVL[0])

  xs = [None] * 3
  mappings = [None] * 3
  xcur = None
  for l in range(DEPTH):
    n, tab = NLVL[l], TAB[l]
    kn, tab_n = NLVL[l + 1], TAB[l + 1]
    xcur, _, keys, g = tc_combine(
        accp, xw, dinv[l], eB[l].reshape(1, C), pP[l].reshape(C, 1),
        n, tab, relu=True, score=True)
    xs[l] = xcur
    mapping = tc_select(keys.reshape(tab // 128, 1, 128), kn, tab).reshape(tab)
    mappings[l] = mapping
    s_l, d_l = edges[l]
    nsrc, ndst, degp, xp = sc_remap(s_l, d_l, mapping, g, tab, tab_n, kn, n)
    edges[l + 1] = (nsrc, ndst)
    dinv[l + 1] = dinv_of(degp, tab_n)
    xw, y = tc_mm_y(xp, eW[l + 1], dinv[l + 1], tab_n)
    accp = sc_msg(nsrc, ndst, y, zeros_tab[:tab_n], tab_n, kn)

  xcur = tc_combine(
      accp, xw, dinv[DEPTH], eB[DEPTH].reshape(1, C),
      pP[0].reshape(C, 1), NLVL[DEPTH], TAB[DEPTH], relu=True, score=False)[0]

  for i in range(DEPTH):
    j = DEPTH - 1 - i
    n, tab = NLVL[j], TAB[j]
    up = sc_upgather(mappings[j], xcur, tab, TAB[j + 1], NLVL[j + 1])
    xw, y = tc_mm_y_add(xs[j], up, dW[i], dinv[j], tab)
    s_l, d_l = edges[j]
    accp = sc_msg(s_l, d_l, y, zeros_tab[:tab], tab, n)
    xcur = tc_combine(
        accp, xw, dinv[j], dB[i].reshape(1, C), pP[0].reshape(C, 1),
        n, tab, relu=(i < DEPTH - 1), score=False)[0]

  return tc_final(xcur)


# speedup vs baseline: 22.5949x; 1.2225x over previous
import functools
import math

import jax
import jax.numpy as jnp
from jax import lax
from jax.experimental import pallas as pl
from jax.experimental.pallas import tpu as pltpu
from jax.experimental.pallas import tpu_sc as plsc

N0 = 10000
E = 320000
C = 64
CLASSES = 40
DEPTH = 3

NLVL = [10000, 5000, 2500, 1250]
TAB = [10240, 5120, 2560, 1280]
RBK = {10240: 512, 5120: 512, 2560: 512, 1280: 640}
RB = [t // 128 for t in TAB]

NC, NS = 2, 16
NW = NC * NS
EPT = 10240
E_PAD = EPT * NW
ECH = EPT // 128
NBUF = 8

_MESH = plsc.VectorSubcoreMesh(
    core_axis_name="c", subcore_axis_name="s", num_cores=NC, num_subcores=NS)

F32 = jnp.float32
I32 = jnp.int32


def _z():
  return jnp.int32(0)

_SC_PARAMS = pltpu.CompilerParams(needs_layout_passes=False, use_tc_tiling_on_sc=False)


def _wid():
  return lax.axis_index("s") * NC + lax.axis_index("c")


def _nchunks(didx, n):
  total = jnp.float32(0.0)
  for q in range(ECH // 16):
    pos = lax.iota(I32, 16) * 128 + q * 2048
    v = plsc.load_gather(didx, [pos])
    total = total + jnp.sum((v < n).astype(F32))
  return total.astype(I32)


def _mm_y_body(x_ref, w_ref, dinv_ref, xw_ref, y_ref):
  xw = jnp.dot(x_ref[...], w_ref[...], preferred_element_type=F32)
  xw_ref[...] = xw
  y_ref[...] = xw * dinv_ref[...]


def tc_mm_y(x, w, dinv, tab):
  rb = RBK[tab]
  r = tab // rb
  din = x.shape[1]
  return pl.pallas_call(
      _mm_y_body,
      grid=(r,),
      in_specs=[
          pl.BlockSpec((rb, din), lambda i: (i, _z())),
          pl.BlockSpec((din, C), lambda i: (_z(), _z())),
          pl.BlockSpec((rb, 1), lambda i: (i, _z())),
      ],
      out_specs=[
          pl.BlockSpec((rb, C), lambda i: (i, _z())),
          pl.BlockSpec((rb, C), lambda i: (i, _z())),
      ],
      out_shape=[
          jax.ShapeDtypeStruct((tab, C), F32),
          jax.ShapeDtypeStruct((tab, C), F32),
      ],
  )(x, w, dinv)


def _mm_y_add_body(res_ref, up_ref, w_ref, dinv_ref, xw_ref, y_ref):
  s = res_ref[...] + up_ref[...]
  xw = jnp.dot(s, w_ref[...], preferred_element_type=F32)
  xw_ref[...] = xw
  y_ref[...] = xw * dinv_ref[...]


def tc_mm_y_add(res, up, w, dinv, tab):
  rb = RBK[tab]
  r = tab // rb
  return pl.pallas_call(
      _mm_y_add_body,
      grid=(r,),
      in_specs=[
          pl.BlockSpec((rb, C), lambda i: (i, _z())),
          pl.BlockSpec((rb, C), lambda i: (i, _z())),
          pl.BlockSpec((C, C), lambda i: (_z(), _z())),
          pl.BlockSpec((rb, 1), lambda i: (i, _z())),
      ],
      out_specs=[
          pl.BlockSpec((rb, C), lambda i: (i, _z())),
          pl.BlockSpec((rb, C), lambda i: (i, _z())),
      ],
      out_shape=[
          jax.ShapeDtypeStruct((tab, C), F32),
          jax.ShapeDtypeStruct((tab, C), F32),
      ],
  )(res, up, w, dinv)


_KEY_PAD = -1073741825


def _combine_body(n, relu, score, rb, acc_ref, xw_ref, dinv_ref, b_ref, p_ref,
                  x_ref, *rest):
  pid = pl.program_id(0)
  msg = acc_ref[0] + acc_ref[1]
  dinv = dinv_ref[...]
  out = msg * dinv + xw_ref[...] * (2.0 * dinv * dinv) + b_ref[...]
  if relu:
    out = jnp.maximum(out, 0.0)
  rows = pid * rb + lax.broadcasted_iota(I32, (rb, 1), 0)
  out = jnp.where(rows < n, out, 0.0)
  x_ref[...] = out
  if score:
    score_ref, key_ref, g_ref = rest
    p = p_ref[...]
    pn = 1.0 / jnp.sqrt(jnp.sum(p * p))
    sc = jnp.tanh(jnp.dot(out, p, preferred_element_type=F32) * pn)
    score_ref[...] = sc
    bits = lax.bitcast_convert_type(sc, I32)
    key = jnp.where(bits >= 0, bits, bits ^ jnp.int32(0x7FFFFFFF))
    key_ref[...] = jnp.where(rows < n, key, jnp.int32(_KEY_PAD))
    g_ref[...] = out * sc


def tc_combine(accp, xw, dinv, b, p, n, tab, relu, score):
  rb = RBK[tab]
  r = tab // rb
  in_specs = [
      pl.BlockSpec((2, rb, C), lambda i: (_z(), i, _z())),
      pl.BlockSpec((rb, C), lambda i: (i, _z())),
      pl.BlockSpec((rb, 1), lambda i: (i, _z())),
      pl.BlockSpec((1, C), lambda i: (_z(), _z())),
      pl.BlockSpec((C, 1), lambda i: (_z(), _z())),
  ]
  out_specs = [pl.BlockSpec((rb, C), lambda i: (i, _z()))]
  out_shape = [jax.ShapeDtypeStruct((tab, C), F32)]
  if score:
    out_specs += [
        pl.BlockSpec((rb, 1), lambda i: (i, _z())),
        pl.BlockSpec((rb, 1), lambda i: (i, _z())),
        pl.BlockSpec((rb, C), lambda i: (i, _z())),
    ]
    out_shape += [
        jax.ShapeDtypeStruct((tab, 1), F32),
        jax.ShapeDtypeStruct((tab, 1), I32),
        jax.ShapeDtypeStruct((tab, C), F32),
    ]
  return pl.pallas_call(
      functools.partial(_combine_body, n, relu, score, rb),
      grid=(r,),
      in_specs=in_specs,
      out_specs=out_specs,
      out_shape=out_shape,
  )(accp, xw, dinv, b, p)


def _select_body(k, kf_ref, krow_ref, map_ref, sm_i, sm_f):
  pid = pl.program_id(0)
  imin = jnp.int32(-2147483648)

  @pl.when(pid == 0)
  def _bisect():
    kf = kf_ref[...]

    def step(it, bacc):
      b = (31 - it).astype(I32)
      cand = bacc | lax.shift_left(jnp.int32(1), b)
      cnt = jnp.sum((kf >= (cand ^ imin)).astype(F32))
      return jnp.where(cnt >= F32(k), cand, bacc)

    bfin = lax.fori_loop(jnp.int32(0), jnp.int32(32), step, jnp.int32(0))
    t = bfin ^ imin
    c1 = jnp.sum((kf > t).astype(F32))
    sm_i[0] = t
    sm_i[1] = jnp.int32(k) - c1.astype(I32)
    sm_f[0] = 0.0
    sm_f[1] = 0.0

  t = sm_i[0]
  budget = sm_i[1].astype(F32)
  row = krow_ref[0]
  strict = (row > t).astype(F32)
  tie = (row == t).astype(F32)
  m2 = jnp.concatenate([strict, tie], axis=0)
  ia = lax.broadcasted_iota(I32, (128, 128), 0)
  ib = lax.broadcasted_iota(I32, (128, 128), 1)
  u = (ia < ib).astype(F32)
  pref = jnp.dot(m2, u, preferred_element_type=F32)
  ps = pref[0:1, :] + sm_f[0]
  pt = pref[1:2, :] + sm_f[1]
  chosen = (strict > 0.0) | ((tie > 0.0) & (pt < budget))
  mp = ps + jnp.minimum(pt, budget)
  map_ref[0] = jnp.where(chosen, mp.astype(I32), jnp.int32(-1))
  sm_f[0] = sm_f[0] + jnp.sum(strict)
  sm_f[1] = sm_f[1] + jnp.sum(tie)


def tc_select(keys2d, k, tab):
  r = tab // 128
  return pl.pallas_call(
      functools.partial(_select_body, k),
      grid=(r,),
      in_specs=[
          pl.BlockSpec((r, 1, 128), lambda i: (_z(), _z(), _z())),
          pl.BlockSpec((1, 1, 128), lambda i: (i, _z(), _z())),
      ],
      out_specs=pl.BlockSpec((1, 1, 128), lambda i: (i, _z(), _z())),
      out_shape=jax.ShapeDtypeStruct((r, 1, 128), I32),
      scratch_shapes=[
          pltpu.SMEM((2,), I32),
          pltpu.SMEM((2,), F32),
      ],
  )(keys2d, keys2d)


def _deg_reduce_body(degp_ref, dinv_ref):
  deg = jnp.sum(degp_ref[...], axis=0, keepdims=True) + 2.0
  dinv_ref[0] = 1.0 / jnp.sqrt(deg)


def tc_deg_reduce(degp, tab):
  r = tab // 128
  return pl.pallas_call(
      _deg_reduce_body,
      grid=(r,),
      in_specs=[pl.BlockSpec((NW, 128), lambda i: (_z(), i))],
      out_specs=pl.BlockSpec((1, 1, 128), lambda i: (i, _z(), _z())),
      out_shape=jax.ShapeDtypeStruct((r, 1, 128), F32),
  )(degp)


def _final_body(z_ref, o_ref):
  logits = z_ref[...][:, :CLASSES]
  m = jnp.max(logits, axis=1, keepdims=True)
  s = jnp.sum(jnp.exp(logits - m), axis=1, keepdims=True)
  o_ref[...] = logits - m - jnp.log(s)


def tc_final(z):
  return pl.pallas_call(
      _final_body,
      grid=(5,),
      in_specs=[pl.BlockSpec((2000, C), lambda i: (i, _z()))],
      out_specs=pl.BlockSpec((2000, CLASSES), lambda i: (i, _z())),
      out_shape=jax.ShapeDtypeStruct((N0, CLASSES), F32),
  )(z)


def sc_deg(dstm, tab):
  def body(dst_hbm, degp_hbm, dbuf, degv):
    w = _wid()
    base = pl.multiple_of(w * EPT, 128)
    pltpu.sync_copy(dst_hbm.at[pl.ds(base, EPT)], dbuf)
    ones = jnp.ones((16,), F32)
    zeros = jnp.zeros((16,), F32)

    def zstep(i, _):
      degv[pl.ds(i * 16, 16)] = zeros
      return jnp.int32(0)

    lax.fori_loop(jnp.int32(0), jnp.int32(tab // 16), zstep, jnp.int32(0))

    def estep(i, _):
      idx = dbuf[pl.ds(i * 16, 16)]
      plsc.addupdate_scatter(degv, [idx], ones)
      return jnp.int32(0)

    lax.fori_loop(jnp.int32(0), jnp.int32(EPT // 16), estep, jnp.int32(0))
    pltpu.sync_copy(degv, degp_hbm.at[w])

  return pl.kernel(
      body,
      out_type=jax.ShapeDtypeStruct((NW, tab), F32),
      mesh=_MESH,
      compiler_params=_SC_PARAMS,
      scratch_types=[
          pltpu.VMEM((EPT,), I32),
          pltpu.VMEM((tab,), F32),
      ],
  )(dstm)


def sc_msg(src, dstm, y, zeros_tab, tab, nvalid):
  def body(src_hbm, dst_hbm, y_hbm, z_hbm, accp_hbm, sidx, didx, dbufs, rows,
           acc, gsem):
    cid = lax.axis_index("c")
    sid = lax.axis_index("s")
    w = _wid()
    ebase = pl.multiple_of(w * EPT, 128)
    pltpu.sync_copy(src_hbm.at[pl.ds(ebase, EPT)], sidx)
    pltpu.sync_copy(dst_hbm.at[pl.ds(ebase, EPT)], didx)
    rz = tab // NS
    zbase = pl.multiple_of(sid * rz, 8)
    pltpu.sync_copy(z_hbm.at[pl.ds(zbase, rz)], acc.at[pl.ds(zbase, rz)])
    plsc.subcore_barrier()
    nch = _nchunks(didx, nvalid)

    def gather(ch, b):
      bi = jnp.int32(b)
      return pltpu.make_async_copy(
          y_hbm.at[sidx.at[pl.ds(ch * 128, 128)]], rows.at[bi], gsem.at[bi])

    for b in range(NBUF):

      @pl.when(jnp.int32(b) < nch)
      def _prime():
        gather(jnp.int32(b), b).start()

    def round_(r, _):
      for b in range(NBUF):
        ch = r * NBUF + b

        @pl.when(ch < nch)
        def _work():
          gather(ch, b).wait()
          for j in range(8):
            dbufs[b, pl.ds(j * 16, 16)] = didx[pl.ds(ch * 128 + j * 16, 16)]
          pltpu.sync_copy(rows.at[jnp.int32(b)], acc.at[dbufs.at[jnp.int32(b)]],
                          add=True)

          @pl.when(ch + NBUF < nch)
          def _next():
            gather(ch + NBUF, b).start()

      return jnp.int32(0)

    rounds = lax.div(nch + jnp.int32(NBUF - 1), jnp.int32(NBUF))
    lax.fori_loop(jnp.int32(0), rounds, round_, jnp.int32(0))
    plsc.subcore_barrier()
    pltpu.sync_copy(acc.at[pl.ds(zbase, rz)],
                    accp_hbm.at[cid, pl.ds(zbase, rz)])

  return pl.kernel(
      body,
      out_type=jax.ShapeDtypeStruct((NC, tab, C), F32),
      mesh=_MESH,
      compiler_params=_SC_PARAMS,
      scratch_types=[
          pltpu.VMEM((EPT,), I32),
          pltpu.VMEM((EPT,), I32),
          pltpu.VMEM((NBUF, 128), I32),
          pltpu.VMEM((NBUF, 128, C), F32),
          pltpu.VMEM_SHARED((tab, C), F32),
          pltpu.SemaphoreType.DMA((NBUF,)),
      ],
  )(src, dstm, y, zeros_tab)


def sc_remap(src, dstm, mapping, g, tab_l, tab_n, kn, nvalid):
  rchunks = tab_l // 128
  rounds = (rchunks + NW - 1) // NW

  def body(src_hbm, dst_hbm, map_hbm, g_hbm, nsrc_hbm, ndst_hbm, degp_hbm,
           xp_hbm, mapv, sidx, didx, osx, odx, ibuf, rows, degv, sem):
    w = _wid()
    ebase = pl.multiple_of(w * EPT, 128)
    pltpu.sync_copy(map_hbm, mapv)
    pltpu.sync_copy(src_hbm.at[pl.ds(ebase, EPT)], sidx)
    pltpu.sync_copy(dst_hbm.at[pl.ds(ebase, EPT)], didx)
    ones = jnp.ones((16,), F32)
    zeros = jnp.zeros((16,), F32)
    zeros_i = jnp.zeros((16,), I32)
    dummy_i = jnp.full((16,), kn, I32)

    def zstep(i, _):
      degv[pl.ds(i * 16, 16)] = zeros
      return jnp.int32(0)

    lax.fori_loop(jnp.int32(0), jnp.int32(tab_n // 16), zstep, jnp.int32(0))

    def pstep(i, _):
      osx[pl.ds(i * 16, 16)] = zeros_i
      odx[pl.ds(i * 16, 16)] = dummy_i
      return jnp.int32(0)

    lax.fori_loop(jnp.int32(0), jnp.int32(EPT // 16), pstep, jnp.int32(0))
    ngroups = _nchunks(didx, nvalid) * 8

    def estep(i, cnt):
      sv = sidx[pl.ds(i * 16, 16)]
      dv = didx[pl.ds(i * 16, 16)]
      ms = plsc.load_gather(mapv, [sv])
      md = plsc.load_gather(mapv, [dv])
      valid = (ms >= 0) & (md >= 0)
      plsc.store_compressed(osx.at[pl.ds(cnt, 16)], ms, mask=valid)
      plsc.store_compressed(odx.at[pl.ds(cnt, 16)], md, mask=valid)
      plsc.addupdate_scatter(degv, [md], ones, mask=valid)
      return cnt + jnp.sum(valid.astype(F32)).astype(I32)

    lax.fori_loop(jnp.int32(0), ngroups, estep, jnp.int32(0))
    pltpu.sync_copy(osx, nsrc_hbm.at[pl.ds(ebase, EPT)])
    pltpu.sync_copy(odx, ndst_hbm.at[pl.ds(ebase, EPT)])
    pltpu.sync_copy(degv, degp_hbm.at[w])

    def rstep(it, _):
      ck = w + it * NW

      @pl.when(ck < rchunks)
      def _do():
        g0 = pl.multiple_of(ck * 128, 128)
        pltpu.sync_copy(g_hbm.at[pl.ds(g0, 128)], rows)
        for j in range(8):
          mv = mapv[pl.ds(g0 + j * 16, 16)]
          ibuf[pl.ds(j * 16, 16)] = jnp.where(mv >= 0, mv, jnp.int32(kn))
        pltpu.async_copy(rows, xp_hbm.at[ibuf], sem).wait()

      return jnp.int32(0)

    lax.fori_loop(jnp.int32(0), jnp.int32(rounds), rstep, jnp.int32(0))

  return pl.kernel(
      body,
      out_type=[
          jax.ShapeDtypeStruct((E_PAD,), I32),
          jax.ShapeDtypeStruct((E_PAD,), I32),
          jax.ShapeDtypeStruct((NW, tab_n), F32),
          jax.ShapeDtypeStruct((tab_n, C), F32),
      ],
      mesh=_MESH,
      compiler_params=_SC_PARAMS,
      scratch_types=[
          pltpu.VMEM((tab_l,), I32),
          pltpu.VMEM((EPT,), I32),
          pltpu.VMEM((EPT,), I32),
          pltpu.VMEM((EPT,), I32),
          pltpu.VMEM((EPT,), I32),
          pltpu.VMEM((128,), I32),
          pltpu.VMEM((128, C), F32),
          pltpu.VMEM((tab_n,), F32),
          pltpu.SemaphoreType.DMA,
      ],
  )(src, dstm, mapping, g)


def sc_upgather(mapping, xnext, tab_j, tab_n, dummy):
  rchunks = tab_j // 128
  rounds = (rchunks + NW - 1) // NW

  def body(map_hbm, xn_hbm, up_hbm, mbuf, ibuf, rows, sem):
    w = _wid()

    def rstep(it, _):
      ck = w + it * NW

      @pl.when(ck < rchunks)
      def _do():
        g0 = pl.multiple_of(ck * 128, 128)
        pltpu.sync_copy(map_hbm.at[pl.ds(g0, 128)], mbuf)
        for j in range(8):
          mv = mbuf[pl.ds(j * 16, 16)]
          ibuf[pl.ds(j * 16, 16)] = jnp.where(mv >= 0, mv, jnp.int32(dummy))
        pltpu.async_copy(xn_hbm.at[ibuf], rows, sem).wait()
        pltpu.sync_copy(rows, up_hbm.at[pl.ds(g0, 128)])

      return jnp.int32(0)

    lax.fori_loop(jnp.int32(0), jnp.int32(rounds), rstep, jnp.int32(0))

  return pl.kernel(
      body,
      out_type=jax.ShapeDtypeStruct((tab_j, C), F32),
      mesh=_MESH,
      compiler_params=_SC_PARAMS,
      scratch_types=[
          pltpu.VMEM((128,), I32),
          pltpu.VMEM((128,), I32),
          pltpu.VMEM((128, C), F32),
          pltpu.SemaphoreType.DMA,
      ],
  )(mapping, xnext)


def kernel(x, edge_index, enc_W0, enc_b0, enc_W1, enc_b1, enc_W2, enc_b2,
           enc_W3, enc_b3, pool_p0, pool_p1, pool_p2, dec_W0, dec_b0, dec_W1,
           dec_b1, dec_W2, dec_b2):
  eW = [enc_W0, enc_W1, enc_W2, enc_W3]
  eB = [enc_b0, enc_b1, enc_b2, enc_b3]
  pP = [pool_p0, pool_p1, pool_p2]
  w2p = jnp.zeros((C, C), F32).at[:, :CLASSES].set(dec_W2)
  b2p = jnp.zeros((C,), F32).at[:CLASSES].set(dec_b2)
  dW = [dec_W0, dec_W1, w2p]
  dB = [dec_b0, dec_b1, b2p]

  src0 = edge_index[0].astype(I32)
  dst0 = edge_index[1].astype(I32)
  src0 = jnp.concatenate(
      [src0, jnp.zeros((E_PAD - E,), I32)])
  dst0 = jnp.concatenate(
      [dst0, jnp.full((E_PAD - E,), NLVL[0], I32)])

  xin = jnp.pad(x.astype(F32), ((0, TAB[0] - N0), (0, 0)))
  zeros_tab = jnp.zeros((TAB[0], C), F32)

  def dinv_of(degp, tab):
    return tc_deg_reduce(degp, tab).reshape(tab, 1)

  dinv = [None] * 4
  dinv[0] = dinv_of(sc_deg(dst0, TAB[0]), TAB[0])
  edges = [(src0, dst0)] + [None] * 3
  xw, y = tc_mm_y(xin, eW[0], dinv[0], TAB[0])
  accp = sc_msg(src0, dst0, y, zeros_tab, TAB[0], NLVL[0])

  xs = [None] * 3
  mappings = [None] * 3
  xcur = None
  for l in range(DEPTH):
    n, tab = NLVL[l], TAB[l]
    kn, tab_n = NLVL[l + 1], TAB[l + 1]
    xcur, _, keys, g = tc_combine(
        accp, xw, dinv[l], eB[l].reshape(1, C), pP[l].reshape(C, 1),
        n, tab, relu=True, score=True)
    xs[l] = xcur
    mapping = tc_select(keys.reshape(tab // 128, 1, 128), kn, tab).reshape(tab)
    mappings[l] = mapping
    s_l, d_l = edges[l]
    nsrc, ndst, degp, xp = sc_remap(s_l, d_l, mapping, g, tab, tab_n, kn, n)
    edges[l + 1] = (nsrc, ndst)
    dinv[l + 1] = dinv_of(degp, tab_n)
    xw, y = tc_mm_y(xp, eW[l + 1], dinv[l + 1], tab_n)
    accp = sc_msg(nsrc, ndst, y, zeros_tab[:tab_n], tab_n, kn)

  xcur = tc_combine(
      accp, xw, dinv[DEPTH], eB[DEPTH].reshape(1, C),
      pP[0].reshape(C, 1), NLVL[DEPTH], TAB[DEPTH], relu=True, score=False)[0]

  for i in range(DEPTH):
    j = DEPTH - 1 - i
    n, tab = NLVL[j], TAB[j]
    up = sc_upgather(mappings[j], xcur, tab, TAB[j + 1], NLVL[j + 1])
    xw, y = tc_mm_y_add(xs[j], up, dW[i], dinv[j], tab)
    s_l, d_l = edges[j]
    accp = sc_msg(s_l, d_l, y, zeros_tab[:tab], tab, n)
    xcur = tc_combine(
        accp, xw, dinv[j], dB[i].reshape(1, C), pP[0].reshape(C, 1),
        n, tab, relu=(i < DEPTH - 1), score=False)[0]

  return tc_final(xcur)
